# baseline (speedup 1.0000x reference)
import functools

import jax
import jax.numpy as jnp
from jax import lax
from jax.experimental import pallas as pl
from jax.experimental.pallas import tpu as pltpu
from jax.experimental.pallas import tpu_sc as plsc

N = 10000
NP = 10240
D = 128
H = 128
E = 320000
LANES = 128
NW = 32
_ch = -(-E // (NW * LANES))
CH = _ch + (_ch % 2)
EP = NW * CH * LANES
STRIPE = NP // 16
BR = 640
TOTCH = NW * CH
N0 = 160
N1 = 2 * CH - N0

_mesh = plsc.VectorSubcoreMesh(core_axis_name="c", subcore_axis_name="s")


@functools.partial(
    pl.kernel,
    out_type=jax.ShapeDtypeStruct((2, NP), jnp.float32),
    mesh=_mesh,
    scratch_types=[
        pltpu.VMEM((CH, LANES), jnp.int32),
        pltpu.VMEM((CH, LANES), jnp.float32),
        pltpu.VMEM_SHARED((NP,), jnp.float32),
    ],
)
def _deg_kernel(colp, ewp, zeros_n, out, col_v, ew_v, deg):
    c = lax.axis_index("c")
    s = lax.axis_index("s")
    wid = s * 2 + c
    rows = pl.ds(s * STRIPE, STRIPE)
    pltpu.sync_copy(zeros_n.at[rows], deg.at[rows])
    pltpu.sync_copy(colp.at[wid], col_v)
    pltpu.sync_copy(ewp.at[wid], ew_v)
    plsc.subcore_barrier()

    def body(j, carry):
        pltpu.sync_copy(ew_v.at[j], deg.at[col_v.at[j]], add=True)
        return carry

    lax.fori_loop(0, CH, body, 0)
    plsc.subcore_barrier()
    pltpu.sync_copy(deg.at[rows], out.at[c, rows])


@functools.partial(
    pl.kernel,
    out_type=jax.ShapeDtypeStruct((2, NP, H), jnp.float32),
    mesh=_mesh,
    scratch_types=[
        pltpu.VMEM((2, LANES), jnp.int32),
        pltpu.VMEM((2, LANES), jnp.int32),
        pltpu.VMEM((2, LANES), jnp.int32),
        pltpu.VMEM((2, LANES), jnp.int32),
        pltpu.VMEM((LANES,), jnp.float32),
        pltpu.VMEM((LANES,), jnp.float32),
        pltpu.VMEM((LANES,), jnp.float32),
        pltpu.VMEM((LANES,), jnp.float32),
        pltpu.VMEM((LANES, H), jnp.float32),
        pltpu.VMEM((LANES, H), jnp.float32),
        pltpu.VMEM_SHARED((NP, H), jnp.float32),
        pltpu.SemaphoreType.DMA,
        pltpu.SemaphoreType.DMA,
        pltpu.SemaphoreType.DMA,
        pltpu.SemaphoreType.DMA,
        pltpu.SemaphoreType.DMA,
        pltpu.SemaphoreType.DMA,
        pltpu.SemaphoreType.DMA,
        pltpu.SemaphoreType.DMA,
    ],
)
def _edge_kernel(hp, ed, ewd, zeros_nh, out, ed0, ed1, ed2, ed3,
                 ew0, ew1, ew2, ew3, buf_a, buf_b,
                 acc, se0, se1, se2, se3, sg0, sg1, ss0, ss1):
    gbufs0 = (buf_a, buf_b)
    c = lax.axis_index("c")
    s = lax.axis_index("s")
    base = jnp.where(c == 0, s * N0, 16 * N0 + s * N1)
    nc = jnp.where(c == 0, N0, N1)
    rows = pl.ds(s * STRIPE, STRIPE)

    @pl.when(c == 0)
    def _():
        pltpu.sync_copy(hp.at[rows], acc.at[rows])

    @pl.when(c == 1)
    def _():
        pltpu.sync_copy(zeros_nh.at[rows], acc.at[rows])

    plsc.subcore_barrier()

    def e_start(j, ebuf, wbuf, sem):
        pltpu.async_copy(ed.at[base + j], ebuf, sem)
        pltpu.async_copy(ewd.at[base + j], wbuf, sem)

    def e_wait(j, ebuf, wbuf, sem):
        pltpu.make_async_copy(ed.at[base + j], ebuf, sem).wait()
        pltpu.make_async_copy(ewd.at[base + j], wbuf, sem).wait()

    def g_start(ebuf, buf, sem):
        pltpu.async_copy(hp.at[ebuf.at[0]], buf, sem)

    def g_wait(buf, sem):
        pltpu.make_async_copy(hp.at[pl.ds(0, LANES)], buf, sem).wait()

    def s_start(ebuf, buf, sem):
        pltpu.async_copy(buf, acc.at[ebuf.at[1]], sem, add=True)

    def s_wait(buf, sem):
        pltpu.make_async_copy(buf, acc.at[pl.ds(0, LANES)], sem).wait()

    def scale(buf, wbuf):
        def scale16(g, carry2):
            wv = wbuf[pl.ds(g * 16, 16)]
            base = g * 16
            for l in range(16):
                w = wv[l]
                i = base + l
                for k in range(H // 16):
                    sl = pl.ds(k * 16, 16)
                    buf[i, sl] = buf[i, sl] * w
            return carry2

        lax.fori_loop(0, LANES // 16, scale16, 0)

    eds = (ed0, ed1, ed2, ed3)
    ews = (ew0, ew1, ew2, ew3)
    ses = (se0, se1, se2, se3)
    gbufs = (buf_a, buf_b)
    sgs = (sg0, sg1)
    sss = (ss0, ss1)
    NI = nc // 4

    @pl.when(nc > 0)
    def _():
        pltpu.sync_copy(ed.at[base], ed0)
        pltpu.sync_copy(ewd.at[base], ew0)
        g_start(ed0, buf_a, sg0)
        e_start(1, ed1, ew1, se1)
        e_start(2, ed2, ew2, se2)

    def step(j, t):
        g_wait(gbufs[t % 2], sgs[t % 2])

        @pl.when(j > 0)
        def _():
            s_wait(gbufs[(t + 1) % 2], sss[(t + 1) % 2])

        @pl.when(j + 3 < nc)
        def _():
            e_start(j + 3, eds[(t + 3) % 4], ews[(t + 3) % 4],
                    ses[(t + 3) % 4])

        @pl.when(j + 1 < nc)
        def _():
            e_wait(j + 1, eds[(t + 1) % 4], ews[(t + 1) % 4],
                   ses[(t + 1) % 4])
            g_start(eds[(t + 1) % 4], gbufs[(t + 1) % 2], sgs[(t + 1) % 2])

        scale(gbufs[t % 2], ews[t])
        s_start(eds[t], gbufs[t % 2], sss[t % 2])

    def pipe(q, carry):
        j0 = 4 * q
        for t in range(4):
            step(j0 + t, t)
        return carry

    @pl.when(nc > 0)
    def _():
        lax.fori_loop(0, NI, pipe, 0)
        s_wait(buf_b, ss1)

    plsc.subcore_barrier()
    pltpu.sync_copy(acc.at[rows], out.at[c, rows])


def _hp1_body(x_ref, w_ref, d0_ref, d1_ref, dis_ref, hp_ref):
    dis = lax.rsqrt(1.0 + d0_ref[...] + d1_ref[...])
    dis_ref[...] = dis
    hp_ref[...] = dis * jnp.dot(x_ref[...], w_ref[...],
                                preferred_element_type=jnp.float32)


def _tc_hp1(xp, W1, d0, d1):
    return pl.pallas_call(
        _hp1_body,
        grid=(NP // BR,),
        in_specs=[
            pl.BlockSpec((BR, D), lambda i: (i, 0)),
            pl.BlockSpec((D, H), lambda i: (0, 0)),
            pl.BlockSpec((BR, 1), lambda i: (i, 0)),
            pl.BlockSpec((BR, 1), lambda i: (i, 0)),
        ],
        out_specs=[
            pl.BlockSpec((BR, 1), lambda i: (i, 0)),
            pl.BlockSpec((BR, H), lambda i: (i, 0)),
        ],
        out_shape=[
            jax.ShapeDtypeStruct((NP, 1), jnp.float32),
            jax.ShapeDtypeStruct((NP, H), jnp.float32),
        ],
    )(xp, W1, d0, d1)


def _mid_body(s0_ref, s1_ref, dis_ref, b_ref, w_ref, hp2_ref):
    z = jnp.maximum(dis_ref[...] * (s0_ref[...] + s1_ref[...]) + b_ref[...],
                    0.0)
    hp2_ref[...] = dis_ref[...] * jnp.dot(z, w_ref[...],
                                          preferred_element_type=jnp.float32)


def _tc_mid(s0, s1, dis, b1, W2):
    return pl.pallas_call(
        _mid_body,
        grid=(NP // BR,),
        in_specs=[
            pl.BlockSpec((BR, H), lambda i: (i, 0)),
            pl.BlockSpec((BR, H), lambda i: (i, 0)),
            pl.BlockSpec((BR, 1), lambda i: (i, 0)),
            pl.BlockSpec((1, H), lambda i: (0, 0)),
            pl.BlockSpec((H, H), lambda i: (0, 0)),
        ],
        out_specs=pl.BlockSpec((BR, H), lambda i: (i, 0)),
        out_shape=jax.ShapeDtypeStruct((NP, H), jnp.float32),
    )(s0, s1, dis, b1, W2)


def _fin_body(s0_ref, s1_ref, dis_ref, b_ref, out_ref):
    out_ref[...] = jnp.maximum(
        dis_ref[...] * (s0_ref[...] + s1_ref[...]) + b_ref[...], 0.0)


def _tc_fin(s0, s1, dis, b2):
    return pl.pallas_call(
        _fin_body,
        grid=(NP // BR,),
        in_specs=[
            pl.BlockSpec((BR, H), lambda i: (i, 0)),
            pl.BlockSpec((BR, H), lambda i: (i, 0)),
            pl.BlockSpec((BR, 1), lambda i: (i, 0)),
            pl.BlockSpec((1, H), lambda i: (0, 0)),
        ],
        out_specs=pl.BlockSpec((BR, H), lambda i: (i, 0)),
        out_shape=jax.ShapeDtypeStruct((NP, H), jnp.float32),
    )(s0, s1, dis, b2)


def kernel(x, edge_index, edge_weight, W1, b1, W2, b2):
    row = edge_index[0]
    col = edge_index[1]
    pe = EP - E
    pad_col = jnp.arange(pe, dtype=col.dtype) % NP
    rowp = jnp.pad(row, (0, pe)).reshape(NW, CH, LANES)
    colp = jnp.concatenate([col, pad_col]).reshape(NW, CH, LANES)
    ewp = jnp.pad(edge_weight, (0, pe)).reshape(NW, CH, LANES)
    ed = jnp.stack([rowp, colp], axis=2).reshape(TOTCH, 2, LANES)
    ewf = ewp.reshape(TOTCH, LANES)
    xp = jnp.pad(x, ((0, NP - N), (0, 0)))
    zn = jnp.zeros((NP,), jnp.float32)
    znh = jnp.zeros((NP, H), jnp.float32)

    degs = _deg_kernel(colp, ewp, zn)
    d0 = degs[0].reshape(NP, 1)
    d1 = degs[1].reshape(NP, 1)
    dis, hp1 = _tc_hp1(xp, W1, d0, d1)

    S = _edge_kernel(hp1, ed, ewf, znh)
    hp2 = _tc_mid(S[0], S[1], dis, b1.reshape(1, H), W2)

    S2 = _edge_kernel(hp2, ed, ewf, znh)
    out = _tc_fin(S2[0], S2[1], dis, b2.reshape(1, H))
    return out[:N]


# speedup vs baseline: 1.5073x; 1.5073x over previous
import functools

import jax
import jax.numpy as jnp
from jax import lax
from jax.experimental import pallas as pl
from jax.experimental.pallas import tpu as pltpu
from jax.experimental.pallas import tpu_sc as plsc

N = 10000
NP = 10240
D = 128
H = 128
E = 320000
LANES = 128
NW = 32
CH = -(-E // (NW * LANES))
EP = NW * CH * LANES
STRIPE = NP // 16
BR = 640

_mesh = plsc.VectorSubcoreMesh(core_axis_name="c", subcore_axis_name="s")


@functools.partial(
    pl.kernel,
    out_type=jax.ShapeDtypeStruct((2, NP), jnp.float32),
    mesh=_mesh,
    scratch_types=[
        pltpu.VMEM((CH, LANES), jnp.int32),
        pltpu.VMEM((CH, LANES), jnp.float32),
        pltpu.VMEM_SHARED((NP,), jnp.float32),
    ],
)
def _deg_kernel(colp, ewp, zeros_n, out, col_v, ew_v, deg):
    c = lax.axis_index("c")
    s = lax.axis_index("s")
    wid = s * 2 + c
    rows = pl.ds(s * STRIPE, STRIPE)
    pltpu.sync_copy(zeros_n.at[rows], deg.at[rows])
    pltpu.sync_copy(colp.at[wid], col_v)
    pltpu.sync_copy(ewp.at[wid], ew_v)
    plsc.subcore_barrier()

    def body(j, carry):
        pltpu.sync_copy(ew_v.at[j], deg.at[col_v.at[j]], add=True)
        return carry

    lax.fori_loop(0, CH, body, 0)
    plsc.subcore_barrier()
    pltpu.sync_copy(deg.at[rows], out.at[c, rows])


@functools.partial(
    pl.kernel,
    out_type=jax.ShapeDtypeStruct((2, NP, H), jnp.float32),
    mesh=_mesh,
    scratch_types=[
        pltpu.VMEM((CH, LANES), jnp.int32),
        pltpu.VMEM((CH, LANES), jnp.int32),
        pltpu.VMEM((CH, LANES), jnp.float32),
        pltpu.VMEM((LANES, H), jnp.float32),
        pltpu.VMEM_SHARED((NP, H), jnp.float32),
        pltpu.SemaphoreType.DMA,
    ],
)
def _edge_kernel(hp, rowp, colp, ewp, zeros_nh, out, row_v, col_v, ew_v,
                 rows_v, acc, sem):
    c = lax.axis_index("c")
    s = lax.axis_index("s")
    wid = s * 2 + c
    rows = pl.ds(s * STRIPE, STRIPE)

    @pl.when(c == 0)
    def _():
        pltpu.sync_copy(hp.at[rows], acc.at[rows])

    @pl.when(c == 1)
    def _():
        pltpu.sync_copy(zeros_nh.at[rows], acc.at[rows])

    pltpu.sync_copy(rowp.at[wid], row_v)
    pltpu.sync_copy(colp.at[wid], col_v)
    pltpu.sync_copy(ewp.at[wid], ew_v)
    plsc.subcore_barrier()

    def chunk(j, carry):
        pltpu.async_copy(hp.at[row_v.at[j]], rows_v, sem).wait()

        def scale16(g, carry2):
            wv = ew_v[j, pl.ds(g * 16, 16)]
            base = g * 16
            for l in range(16):
                w = wv[l]
                i = base + l
                for k in range(H // 16):
                    sl = pl.ds(k * 16, 16)
                    rows_v[i, sl] = rows_v[i, sl] * w
            return carry2

        lax.fori_loop(0, LANES // 16, scale16, 0)
        pltpu.sync_copy(rows_v, acc.at[col_v.at[j]], add=True)
        return carry

    lax.fori_loop(0, CH, chunk, 0)
    plsc.subcore_barrier()
    pltpu.sync_copy(acc.at[rows], out.at[c, rows])


def _hp1_body(x_ref, w_ref, d0_ref, d1_ref, dis_ref, hp_ref):
    dis = lax.rsqrt(1.0 + d0_ref[...] + d1_ref[...])
    dis_ref[...] = dis
    hp_ref[...] = dis * jnp.dot(x_ref[...], w_ref[...],
                                preferred_element_type=jnp.float32)


def _tc_hp1(xp, W1, d0, d1):
    return pl.pallas_call(
        _hp1_body,
        grid=(NP // BR,),
        in_specs=[
            pl.BlockSpec((BR, D), lambda i: (i, 0)),
            pl.BlockSpec((D, H), lambda i: (0, 0)),
            pl.BlockSpec((BR, 1), lambda i: (i, 0)),
            pl.BlockSpec((BR, 1), lambda i: (i, 0)),
        ],
        out_specs=[
            pl.BlockSpec((BR, 1), lambda i: (i, 0)),
            pl.BlockSpec((BR, H), lambda i: (i, 0)),
        ],
        out_shape=[
            jax.ShapeDtypeStruct((NP, 1), jnp.float32),
            jax.ShapeDtypeStruct((NP, H), jnp.float32),
        ],
    )(xp, W1, d0, d1)


def _mid_body(s0_ref, s1_ref, dis_ref, b_ref, w_ref, hp2_ref):
    z = jnp.maximum(dis_ref[...] * (s0_ref[...] + s1_ref[...]) + b_ref[...],
                    0.0)
    hp2_ref[...] = dis_ref[...] * jnp.dot(z, w_ref[...],
                                          preferred_element_type=jnp.float32)


def _tc_mid(s0, s1, dis, b1, W2):
    return pl.pallas_call(
        _mid_body,
        grid=(NP // BR,),
        in_specs=[
            pl.BlockSpec((BR, H), lambda i: (i, 0)),
            pl.BlockSpec((BR, H), lambda i: (i, 0)),
            pl.BlockSpec((BR, 1), lambda i: (i, 0)),
            pl.BlockSpec((1, H), lambda i: (0, 0)),
            pl.BlockSpec((H, H), lambda i: (0, 0)),
        ],
        out_specs=pl.BlockSpec((BR, H), lambda i: (i, 0)),
        out_shape=jax.ShapeDtypeStruct((NP, H), jnp.float32),
    )(s0, s1, dis, b1, W2)


def _fin_body(s0_ref, s1_ref, dis_ref, b_ref, out_ref):
    out_ref[...] = jnp.maximum(
        dis_ref[...] * (s0_ref[...] + s1_ref[...]) + b_ref[...], 0.0)


def _tc_fin(s0, s1, dis, b2):
    return pl.pallas_call(
        _fin_body,
        grid=(NP // BR,),
        in_specs=[
            pl.BlockSpec((BR, H), lambda i: (i, 0)),
            pl.BlockSpec((BR, H), lambda i: (i, 0)),
            pl.BlockSpec((BR, 1), lambda i: (i, 0)),
            pl.BlockSpec((1, H), lambda i: (0, 0)),
        ],
        out_specs=pl.BlockSpec((BR, H), lambda i: (i, 0)),
        out_shape=jax.ShapeDtypeStruct((NP, H), jnp.float32),
    )(s0, s1, dis, b2)


def kernel(x, edge_index, edge_weight, W1, b1, W2, b2):
    row = edge_index[0]
    col = edge_index[1]
    pe = EP - E
    pad_col = jnp.arange(pe, dtype=col.dtype) % NP
    rowp = jnp.pad(row, (0, pe)).reshape(NW, CH, LANES)
    colp = jnp.concatenate([col, pad_col]).reshape(NW, CH, LANES)
    ewp = jnp.pad(edge_weight, (0, pe)).reshape(NW, CH, LANES)
    xp = jnp.pad(x, ((0, NP - N), (0, 0)))
    zn = jnp.zeros((NP,), jnp.float32)
    znh = jnp.zeros((NP, H), jnp.float32)

    degs = _deg_kernel(colp, ewp, zn)
    d0 = degs[0].reshape(NP, 1)
    d1 = degs[1].reshape(NP, 1)
    dis, hp1 = _tc_hp1(xp, W1, d0, d1)

    S = _edge_kernel(hp1, rowp, colp, ewp, znh)
    hp2 = _tc_mid(S[0], S[1], dis, b1.reshape(1, H), W2)

    S2 = _edge_kernel(hp2, rowp, colp, ewp, znh)
    out = _tc_fin(S2[0], S2[1], dis, b2.reshape(1, H))
    return out[:N]
